# initial kernel scaffold (unmeasured)
import jax
import jax.numpy as jnp
from jax import lax
from jax.experimental import pallas as pl
from jax.experimental.pallas import tpu as pltpu


def kernel(
    t,
):
    def body(*refs):
        pass

    out_shape = jax.ShapeDtypeStruct(..., jnp.float32)
    return pl.pallas_call(body, out_shape=out_shape)(...)


# baseline (device time: 218006 ns/iter reference)
import jax
import jax.numpy as jnp
from jax import lax
from jax.experimental import pallas as pl
from jax.experimental.pallas import tpu as pltpu

M, N = 2048, 1024
N_ROUNDS = 5
HALF = [1024, 512, 256, 128, 64]
COMM_OFF = [0, 1024, 1536, 1792, 1920]


def _coords(d):
    z = d // 8
    s8 = d % 8
    x = ((s8 + 1) >> 1) & 1
    y = s8 >> 1
    return x, y, z


def _logical_id(x, y, z):
    return 8 * z + 2 * y + (x ^ (y & 1))


def kernel(t):
    def body(x_ref, out_ref, comm_ref, send_sems, recv_sems):
        d = lax.axis_index("i")
        x, y, z = _coords(d)

        partners = [
            _logical_id(1 - x, y, z),
            _logical_id(x, y ^ 1, z),
            _logical_id(x, y, z ^ 1),
            _logical_id(x, y ^ 2, z),
            _logical_id(x, y, z ^ 2),
        ]
        bits = [x, y & 1, z & 1, (y >> 1) & 1, (z >> 1) & 1]

        barrier_sem = pltpu.get_barrier_semaphore()
        for p in partners:
            pl.semaphore_signal(
                barrier_sem, inc=1,
                device_id=(p,), device_id_type=pl.DeviceIdType.MESH,
            )
        pl.semaphore_wait(barrier_sem, N_ROUNDS)

        out_ref[:, :] = x_ref[:, :]

        o = jnp.int32(0)
        for r in range(N_ROUNDS):
            half = HALF[r]
            bit = bits[r]
            send_off = o + jnp.where(bit == 0, half, 0)
            keep_off = o + jnp.where(bit == 0, 0, half)
            rdma = pltpu.make_async_remote_copy(
                src_ref=out_ref.at[pl.ds(pl.multiple_of(send_off, 64), half), :],
                dst_ref=comm_ref.at[pl.ds(COMM_OFF[r], half), :],
                send_sem=send_sems.at[r],
                recv_sem=recv_sems.at[r],
                device_id=(partners[r],),
                device_id_type=pl.DeviceIdType.MESH,
            )
            rdma.start()
            rdma.wait()
            out_ref[pl.ds(pl.multiple_of(keep_off, 64), half), :] = (
                out_ref[pl.ds(pl.multiple_of(keep_off, 64), half), :]
                + comm_ref[pl.ds(COMM_OFF[r], half), :]
            )
            o = keep_off

        s = out_ref[pl.ds(pl.multiple_of(o, 64), 64), :]
        rel = jnp.maximum(s, 0.0)
        out_ref[pl.ds(pl.multiple_of(o, 64), 64), :] = jnp.tanh(s) * s * s + rel * rel * rel

        sz = 64
        for r in reversed(range(N_ROUNDS)):
            rdma = pltpu.make_async_remote_copy(
                src_ref=out_ref.at[pl.ds(pl.multiple_of(o, 64), sz), :],
                dst_ref=out_ref.at[pl.ds(pl.multiple_of(o, 64), sz), :],
                send_sem=send_sems.at[N_ROUNDS + r],
                recv_sem=recv_sems.at[N_ROUNDS + r],
                device_id=(partners[r],),
                device_id_type=pl.DeviceIdType.MESH,
            )
            rdma.start()
            rdma.wait()
            o = o - bits[r] * sz
            sz *= 2

    return pl.pallas_call(
        body,
        out_shape=jax.ShapeDtypeStruct((M, N), jnp.float32),
        in_specs=[pl.BlockSpec(memory_space=pltpu.VMEM)],
        out_specs=pl.BlockSpec(memory_space=pltpu.VMEM),
        scratch_shapes=[
            pltpu.VMEM((1984, N), jnp.float32),
            pltpu.SemaphoreType.DMA((2 * N_ROUNDS,)),
            pltpu.SemaphoreType.DMA((2 * N_ROUNDS,)),
        ],
        compiler_params=pltpu.CompilerParams(collective_id=0),
    )(t)


# device time: 124060 ns/iter; 1.7573x vs baseline; 1.7573x over previous
import jax
import jax.numpy as jnp
from jax import lax
from jax.experimental import pallas as pl
from jax.experimental.pallas import tpu as pltpu

M, N = 2048, 1024
N_ROUNDS = 5
HALF = [1024, 512, 256, 128, 64]
COMM_OFF = [0, 1024, 1536, 1792, 1920]


def _coords(d):
    z = d // 8
    s8 = d % 8
    x = ((s8 + 1) >> 1) & 1
    y = s8 >> 1
    return x, y, z


def _logical_id(x, y, z):
    return 8 * z + 2 * y + (x ^ (y & 1))


def _mo(off):
    return pl.multiple_of(off, 64)


def kernel(t):
    def body(x_ref, out_ref, comm_ref, stage_ref, g_ref, send_sems, recv_sems):
        d = lax.axis_index("i")
        x, y, z = _coords(d)

        partners = [
            _logical_id(1 - x, y, z),
            _logical_id(x, y ^ 1, z),
            _logical_id(x, y, z ^ 1),
            _logical_id(x, y ^ 2, z),
            _logical_id(x, y, z ^ 2),
        ]
        bits = [x, y & 1, z & 1, (y >> 1) & 1, (z >> 1) & 1]

        barrier_sem = pltpu.get_barrier_semaphore()
        for p in partners:
            pl.semaphore_signal(
                barrier_sem, inc=1,
                device_id=(p,), device_id_type=pl.DeviceIdType.MESH,
            )
        pl.semaphore_wait(barrier_sem, N_ROUNDS)

        out_ref[:, :] = x_ref[:, :]

        o = jnp.int32(0)
        for r in range(N_ROUNDS):
            half = HALF[r]
            bit = bits[r]
            send_off = o + jnp.where(bit == 0, half, 0)
            keep_off = o + jnp.where(bit == 0, 0, half)
            c = COMM_OFF[r]
            stage_ref[pl.ds(c, half), :] = out_ref[
                pl.ds(_mo(send_off), half), :
            ].astype(jnp.bfloat16)
            rdma = pltpu.make_async_remote_copy(
                src_ref=stage_ref.at[pl.ds(c, half), :],
                dst_ref=comm_ref.at[pl.ds(c, half), :],
                send_sem=send_sems.at[r],
                recv_sem=recv_sems.at[r],
                device_id=(partners[r],),
                device_id_type=pl.DeviceIdType.MESH,
            )
            rdma.start()
            rdma.wait()
            out_ref[pl.ds(_mo(keep_off), half), :] = (
                out_ref[pl.ds(_mo(keep_off), half), :]
                + comm_ref[pl.ds(c, half), :].astype(jnp.float32)
            )
            o = keep_off

        s = out_ref[pl.ds(_mo(o), 64), :]
        rel = jnp.maximum(s, 0.0)
        fs = jnp.tanh(s) * s * s + rel * rel * rel
        out_ref[pl.ds(_mo(o), 64), :] = fs
        g_ref[pl.ds(_mo(o), 64), :] = fs.astype(jnp.bfloat16)

        sz = 64
        for r in reversed(range(N_ROUNDS)):
            rdma = pltpu.make_async_remote_copy(
                src_ref=g_ref.at[pl.ds(_mo(o), sz), :],
                dst_ref=g_ref.at[pl.ds(_mo(o), sz), :],
                send_sem=send_sems.at[N_ROUNDS + r],
                recv_sem=recv_sems.at[N_ROUNDS + r],
                device_id=(partners[r],),
                device_id_type=pl.DeviceIdType.MESH,
            )
            rdma.start()
            rdma.wait()
            parent = o - bits[r] * sz
            recv_off = 2 * parent + sz - o
            out_ref[pl.ds(_mo(recv_off), sz), :] = g_ref[
                pl.ds(_mo(recv_off), sz), :
            ].astype(jnp.float32)
            o = parent
            sz *= 2

    return pl.pallas_call(
        body,
        out_shape=jax.ShapeDtypeStruct((M, N), jnp.float32),
        in_specs=[pl.BlockSpec(memory_space=pltpu.VMEM)],
        out_specs=pl.BlockSpec(memory_space=pltpu.VMEM),
        scratch_shapes=[
            pltpu.VMEM((1984, N), jnp.bfloat16),
            pltpu.VMEM((1984, N), jnp.bfloat16),
            pltpu.VMEM((M, N), jnp.bfloat16),
            pltpu.SemaphoreType.DMA((2 * N_ROUNDS,)),
            pltpu.SemaphoreType.DMA((2 * N_ROUNDS,)),
        ],
        compiler_params=pltpu.CompilerParams(collective_id=0),
    )(t)


# device time: 78421 ns/iter; 2.7799x vs baseline; 1.5820x over previous
import jax
import jax.numpy as jnp
from jax import lax
from jax.experimental import pallas as pl
from jax.experimental.pallas import tpu as pltpu

M, N = 2048, 1024
N_ROUNDS = 5
N_PARTS = 2
PM = M // N_PARTS
HALF = [512, 256, 128, 64, 32]
COMM_OFF = [0, 512, 768, 896, 960]
COMM_ROWS = 992


def _coords(d):
    z = d // 8
    s8 = d % 8
    x = ((s8 + 1) >> 1) & 1
    y = s8 >> 1
    return x, y, z


def _logical_id(x, y, z):
    return 8 * z + 2 * y + (x ^ (y & 1))


def _mo(off):
    return pl.multiple_of(off, 32)


def kernel(t):
    def body(x_ref, out_ref, comm_ref, stage_ref, g_ref, send_sems, recv_sems):
        d = lax.axis_index("i")
        x, y, z = _coords(d)

        dim_x = (_logical_id(1 - x, y, z), x)
        dim_y0 = (_logical_id(x, y ^ 1, z), y & 1)
        dim_y1 = (_logical_id(x, y ^ 2, z), (y >> 1) & 1)
        dim_z0 = (_logical_id(x, y, z ^ 1), z & 1)
        dim_z1 = (_logical_id(x, y, z ^ 2), (z >> 1) & 1)
        parts = [
            [dim_x, dim_y0, dim_z0, dim_z1, dim_y1],
            [dim_y0, dim_z0, dim_x, dim_y1, dim_z1],
        ]

        barrier_sem = pltpu.get_barrier_semaphore()
        for p, _ in parts[0]:
            pl.semaphore_signal(
                barrier_sem, inc=1,
                device_id=(p,), device_id_type=pl.DeviceIdType.MESH,
            )
        pl.semaphore_wait(barrier_sem, N_ROUNDS)

        out_ref[:, :] = x_ref[:, :]

        def sem_idx(part, r, ag):
            return (2 * part + ag) * N_ROUNDS + r

        def rs_rdma(part, r, o):
            partner, bit = parts[part][r]
            half = HALF[r]
            send_off = o + jnp.where(bit == 0, half, 0)
            keep_off = o + jnp.where(bit == 0, 0, half)
            c = part * COMM_ROWS + COMM_OFF[r]
            stage_ref[pl.ds(c, half), :] = out_ref[
                pl.ds(_mo(send_off), half), :
            ].astype(jnp.bfloat16)
            rdma = pltpu.make_async_remote_copy(
                src_ref=stage_ref.at[pl.ds(c, half), :],
                dst_ref=comm_ref.at[pl.ds(c, half), :],
                send_sem=send_sems.at[sem_idx(part, r, 0)],
                recv_sem=recv_sems.at[sem_idx(part, r, 0)],
                device_id=(partner,),
                device_id_type=pl.DeviceIdType.MESH,
            )
            rdma.start()
            return rdma, keep_off

        def rs_accum(part, r, keep_off):
            half = HALF[r]
            c = part * COMM_ROWS + COMM_OFF[r]
            out_ref[pl.ds(_mo(keep_off), half), :] = (
                out_ref[pl.ds(_mo(keep_off), half), :]
                + comm_ref[pl.ds(c, half), :].astype(jnp.float32)
            )

        offs = [jnp.int32(part * PM) for part in range(N_PARTS)]
        for r in range(N_ROUNDS):
            started = []
            for part in range(N_PARTS):
                rdma, keep_off = rs_rdma(part, r, offs[part])
                started.append(rdma)
                offs[part] = keep_off
            for part in range(N_PARTS):
                started[part].wait()
                rs_accum(part, r, offs[part])

        for part in range(N_PARTS):
            s = out_ref[pl.ds(_mo(offs[part]), 32), :]
            rel = jnp.maximum(s, 0.0)
            fs = jnp.tanh(s) * s * s + rel * rel * rel
            out_ref[pl.ds(_mo(offs[part]), 32), :] = fs
            g_ref[pl.ds(_mo(offs[part]), 32), :] = fs.astype(jnp.bfloat16)

        sz = PM // 32
        for r in reversed(range(N_ROUNDS)):
            started = []
            for part in range(N_PARTS):
                partner, _ = parts[part][r]
                rdma = pltpu.make_async_remote_copy(
                    src_ref=g_ref.at[pl.ds(_mo(offs[part]), sz), :],
                    dst_ref=g_ref.at[pl.ds(_mo(offs[part]), sz), :],
                    send_sem=send_sems.at[sem_idx(part, r, 1)],
                    recv_sem=recv_sems.at[sem_idx(part, r, 1)],
                    device_id=(partner,),
                    device_id_type=pl.DeviceIdType.MESH,
                )
                rdma.start()
                started.append(rdma)
            for part in range(N_PARTS):
                _, bit = parts[part][r]
                started[part].wait()
                parent = offs[part] - bit * sz
                recv_off = 2 * parent + sz - offs[part]
                out_ref[pl.ds(_mo(recv_off), sz), :] = g_ref[
                    pl.ds(_mo(recv_off), sz), :
                ].astype(jnp.float32)
                offs[part] = parent
            sz *= 2

    return pl.pallas_call(
        body,
        out_shape=jax.ShapeDtypeStruct((M, N), jnp.float32),
        in_specs=[pl.BlockSpec(memory_space=pltpu.VMEM)],
        out_specs=pl.BlockSpec(memory_space=pltpu.VMEM),
        scratch_shapes=[
            pltpu.VMEM((N_PARTS * COMM_ROWS, N), jnp.bfloat16),
            pltpu.VMEM((N_PARTS * COMM_ROWS, N), jnp.bfloat16),
            pltpu.VMEM((M, N), jnp.bfloat16),
            pltpu.SemaphoreType.DMA((4 * N_ROUNDS,)),
            pltpu.SemaphoreType.DMA((4 * N_ROUNDS,)),
        ],
        compiler_params=pltpu.CompilerParams(collective_id=0),
    )(t)
